# Initial kernel scaffold; baseline (speedup 1.0000x reference)
import jax
import jax.numpy as jnp
from jax.experimental import pallas as pl


def kernel(inputs, edgeconv_w, bn1_gamma, bn1_beta, enc_w, bn2_gamma, bn2_beta, wr, wi, wj, wk, quat_bias, cls_w, cls_b):
    raise NotImplementedError("write your pallas kernel here")


# speedup vs baseline: 2.0202x; 2.0202x over previous
import functools
import math

import jax
import jax.numpy as jnp
from jax import lax
from jax.experimental import pallas as pl
from jax.experimental.pallas import tpu as pltpu

_B, _N, _K = 16, 2048, 20
_H1, _H2 = 64, 128
_Q = _H2 // 4
_EPS = 1e-5
_R = 256

_HIGH = lax.Precision.HIGHEST


def _dot(a, b):
    return lax.dot_general(a, b, (((1,), (0,)), ((), ())),
                           precision=_HIGH, preferred_element_type=jnp.float32)


def _fused_kernel(ptsT_ref, ptsR_ref, ptsN_ref, at_ref, ct_ref, g1_ref, b1_ref,
                  encT_ref, g2_ref, b2_ref, wbigT_ref, qb_ref, ssel_ref,
                  clsT_ref, clsb_ref, out_ref, amax_ref, asum_ref):
    t = pl.program_id(1)
    n_t = pl.num_programs(1)

    ptsT = ptsT_ref[0]
    Xi = ptsR_ref[0]
    ptsN = ptsN_ref[0]

    xxj = jnp.sum(ptsT * ptsT, axis=0, keepdims=True)
    xxi = jnp.sum(Xi * Xi, axis=1, keepdims=True)
    inner = -2.0 * lax.dot_general(
        Xi.astype(jnp.bfloat16), ptsT.astype(jnp.bfloat16),
        (((1,), (0,)), ((), ())), preferred_element_type=jnp.float32)
    P = -xxi - inner - xxj

    lane = lax.broadcasted_iota(jnp.int32, (_R, _N), 1)
    neg = jnp.float32(-jnp.inf)
    Mx = jnp.full((_R, _H1), -jnp.inf, jnp.float32)
    Mn = jnp.full((_R, _H1), jnp.inf, jnp.float32)
    at = at_ref[...]
    for _ in range(_K):
        m = jnp.max(P, axis=1, keepdims=True)
        am = jnp.min(jnp.where(P >= m, lane, _N), axis=1, keepdims=True)
        sel = lane == am
        g = _dot(sel.astype(jnp.float32), ptsN)
        u = _dot(g, at)
        Mx = jnp.maximum(Mx, u)
        Mn = jnp.minimum(Mn, u)
        P = jnp.where(sel, neg, P)

    Vi = _dot(Xi, ct_ref[...])
    g1 = g1_ref[...]
    zx = g1 * (Mx + Vi) + b1_ref[...]
    zn = g1 * (Mn + Vi) + b1_ref[...]
    z = jnp.where(g1 >= 0.0, zx, zn)
    edge = jnp.where(z > 0.0, z, 0.2 * z)

    e = g2_ref[...] * _dot(edge, encT_ref[...]) + b2_ref[...]
    e = 0.5 * e * (1.0 + lax.erf(e * (1.0 / math.sqrt(2.0))))

    mixed = _dot(e, wbigT_ref[...]) + qb_ref[...]
    merged = _dot(mixed * mixed, ssel_ref[...])

    pmax = jnp.max(merged, axis=0, keepdims=True)
    psum = jnp.sum(merged, axis=0, keepdims=True)

    @pl.when(t == 0)
    def _():
        amax_ref[...] = pmax
        asum_ref[...] = psum

    @pl.when(t > 0)
    def _():
        amax_ref[...] = jnp.maximum(amax_ref[...], pmax)
        asum_ref[...] = asum_ref[...] + psum

    @pl.when(t == n_t - 1)
    def _():
        fmax = amax_ref[...]
        fmean = asum_ref[...] * (1.0 / _N)
        clsT = clsT_ref[...]
        logits = _dot(fmax, clsT[:_Q]) + _dot(fmean, clsT[_Q:]) + clsb_ref[...]
        out_ref[0] = logits


def kernel(inputs, edgeconv_w, bn1_gamma, bn1_beta, enc_w, bn2_gamma, bn2_beta,
           wr, wi, wj, wk, quat_bias, cls_w, cls_b):
    b = inputs.shape[0]
    points = inputs[..., :4].reshape(b, -1, 4)
    ptsT = jnp.transpose(points, (0, 2, 1))

    s1 = 1.0 / math.sqrt(1.0 + _EPS)
    at = edgeconv_w[:, :4].T
    ct = (edgeconv_w[:, 4:] - edgeconv_w[:, :4]).T
    g1 = (bn1_gamma * s1).reshape(1, _H1)
    b1 = bn1_beta.reshape(1, _H1)
    encT = enc_w.T
    g2 = (bn2_gamma * s1).reshape(1, _H2)
    b2 = bn2_beta.reshape(1, _H2)

    wbig = jnp.block([
        [wr, -wi, -wj, -wk],
        [wi, wr, wk, -wj],
        [wj, -wk, wr, wi],
        [wk, wj, -wi, wr],
    ])
    wbigT = wbig.T
    qb = quat_bias.reshape(1, _H2)

    ch = jnp.arange(_H2)
    ssel = (ch[:, None] // 4 == jnp.arange(_Q)[None, :]).astype(jnp.float32)

    clsT = cls_w.T
    clsb = cls_b.reshape(1, -1)

    n_t = _N // _R
    grid = (b, n_t)
    const = lambda *s: pl.BlockSpec(s, lambda bb, tt: (0,) * len(s))

    out = pl.pallas_call(
        _fused_kernel,
        grid=grid,
        in_specs=[
            pl.BlockSpec((1, 4, _N), lambda bb, tt: (bb, 0, 0)),
            pl.BlockSpec((1, _R, 4), lambda bb, tt: (bb, tt, 0)),
            pl.BlockSpec((1, _N, 4), lambda bb, tt: (bb, 0, 0)),
            const(4, _H1), const(4, _H1), const(1, _H1), const(1, _H1),
            const(_H1, _H2), const(1, _H2), const(1, _H2),
            const(_H2, _H2), const(1, _H2), const(_H2, _Q),
            const(2 * _Q, cls_w.shape[0]), const(1, cls_w.shape[0]),
        ],
        out_specs=pl.BlockSpec((1, 1, cls_w.shape[0]), lambda bb, tt: (bb, 0, 0)),
        out_shape=jax.ShapeDtypeStruct((b, 1, cls_w.shape[0]), jnp.float32),
        scratch_shapes=[
            pltpu.VMEM((1, _Q), jnp.float32),
            pltpu.VMEM((1, _Q), jnp.float32),
        ],
        compiler_params=pltpu.CompilerParams(
            dimension_semantics=("arbitrary", "arbitrary"),
        ),
    )(ptsT, points, points, at, ct, g1, b1, encT, g2, b2, wbigT, qb, ssel,
      clsT, clsb)
    return out.reshape(b, cls_w.shape[0])


# speedup vs baseline: 2.1330x; 1.0558x over previous
import functools
import math

import jax
import jax.numpy as jnp
from jax import lax
from jax.experimental import pallas as pl
from jax.experimental.pallas import tpu as pltpu

_B, _N, _K = 16, 2048, 20
_H1, _H2 = 64, 128
_Q = _H2 // 4
_EPS = 1e-5
_R = 256

_HIGH = lax.Precision.HIGHEST


def _dot(a, b):
    return lax.dot_general(a, b, (((1,), (0,)), ((), ())),
                           precision=_HIGH, preferred_element_type=jnp.float32)


def _fused_kernel(ptsT_ref, ptsR_ref, ptsN_ref, at_ref, ct_ref, g1_ref, b1_ref,
                  encT_ref, g2_ref, b2_ref, wbigT_ref, qb_ref, ssel_ref,
                  clsT_ref, clsb_ref, out_ref, amax_ref, asum_ref):
    t = pl.program_id(1)
    n_t = pl.num_programs(1)

    ptsT = ptsT_ref[0]
    Xi = ptsR_ref[0]
    ptsN = ptsN_ref[0]

    xxj = jnp.sum(ptsT * ptsT, axis=0, keepdims=True)
    xxi = jnp.sum(Xi * Xi, axis=1, keepdims=True)
    inner = -2.0 * lax.dot_general(
        Xi.astype(jnp.bfloat16), ptsT.astype(jnp.bfloat16),
        (((1,), (0,)), ((), ())), preferred_element_type=jnp.float32)
    P = -xxi - inner - xxj

    lane = lax.broadcasted_iota(jnp.int32, (_R, _N), 1)
    neg = jnp.float32(-jnp.inf)
    rid = lax.broadcasted_iota(jnp.int32, (_R, _N), 0) + t * _R
    P = jnp.where(lane == rid, neg, P)
    at = at_ref[...]
    u0 = _dot(Xi, at)
    Mx = u0
    Mn = u0
    for _ in range(_K - 1):
        am = jnp.argmax(P, axis=1)[:, None]
        sel = lane == am
        g = _dot(jnp.where(sel, 1.0, 0.0), ptsN)
        u = _dot(g, at)
        Mx = jnp.maximum(Mx, u)
        Mn = jnp.minimum(Mn, u)
        P = jnp.where(sel, neg, P)

    Vi = _dot(Xi, ct_ref[...])
    g1 = g1_ref[...]
    zx = g1 * (Mx + Vi) + b1_ref[...]
    zn = g1 * (Mn + Vi) + b1_ref[...]
    z = jnp.where(g1 >= 0.0, zx, zn)
    edge = jnp.where(z > 0.0, z, 0.2 * z)

    e = g2_ref[...] * _dot(edge, encT_ref[...]) + b2_ref[...]
    e = 0.5 * e * (1.0 + lax.erf(e * (1.0 / math.sqrt(2.0))))

    mixed = _dot(e, wbigT_ref[...]) + qb_ref[...]
    merged = _dot(mixed * mixed, ssel_ref[...])

    pmax = jnp.max(merged, axis=0, keepdims=True)
    psum = jnp.sum(merged, axis=0, keepdims=True)

    @pl.when(t == 0)
    def _():
        amax_ref[...] = pmax
        asum_ref[...] = psum

    @pl.when(t > 0)
    def _():
        amax_ref[...] = jnp.maximum(amax_ref[...], pmax)
        asum_ref[...] = asum_ref[...] + psum

    @pl.when(t == n_t - 1)
    def _():
        fmax = amax_ref[...]
        fmean = asum_ref[...] * (1.0 / _N)
        clsT = clsT_ref[...]
        logits = _dot(fmax, clsT[:_Q]) + _dot(fmean, clsT[_Q:]) + clsb_ref[...]
        out_ref[0] = logits


def kernel(inputs, edgeconv_w, bn1_gamma, bn1_beta, enc_w, bn2_gamma, bn2_beta,
           wr, wi, wj, wk, quat_bias, cls_w, cls_b):
    b = inputs.shape[0]
    points = inputs[..., :4].reshape(b, -1, 4)
    ptsT = jnp.transpose(points, (0, 2, 1))

    s1 = 1.0 / math.sqrt(1.0 + _EPS)
    at = edgeconv_w[:, :4].T
    ct = (edgeconv_w[:, 4:] - edgeconv_w[:, :4]).T
    g1 = (bn1_gamma * s1).reshape(1, _H1)
    b1 = bn1_beta.reshape(1, _H1)
    encT = enc_w.T
    g2 = (bn2_gamma * s1).reshape(1, _H2)
    b2 = bn2_beta.reshape(1, _H2)

    wbig = jnp.block([
        [wr, -wi, -wj, -wk],
        [wi, wr, wk, -wj],
        [wj, -wk, wr, wi],
        [wk, wj, -wi, wr],
    ])
    wbigT = wbig.T
    qb = quat_bias.reshape(1, _H2)

    ch = jnp.arange(_H2)
    ssel = (ch[:, None] // 4 == jnp.arange(_Q)[None, :]).astype(jnp.float32)

    clsT = cls_w.T
    clsb = cls_b.reshape(1, -1)

    n_t = _N // _R
    grid = (b, n_t)
    const = lambda *s: pl.BlockSpec(s, lambda bb, tt: (0,) * len(s))

    out = pl.pallas_call(
        _fused_kernel,
        grid=grid,
        in_specs=[
            pl.BlockSpec((1, 4, _N), lambda bb, tt: (bb, 0, 0)),
            pl.BlockSpec((1, _R, 4), lambda bb, tt: (bb, tt, 0)),
            pl.BlockSpec((1, _N, 4), lambda bb, tt: (bb, 0, 0)),
            const(4, _H1), const(4, _H1), const(1, _H1), const(1, _H1),
            const(_H1, _H2), const(1, _H2), const(1, _H2),
            const(_H2, _H2), const(1, _H2), const(_H2, _Q),
            const(2 * _Q, cls_w.shape[0]), const(1, cls_w.shape[0]),
        ],
        out_specs=pl.BlockSpec((1, 1, cls_w.shape[0]), lambda bb, tt: (bb, 0, 0)),
        out_shape=jax.ShapeDtypeStruct((b, 1, cls_w.shape[0]), jnp.float32),
        scratch_shapes=[
            pltpu.VMEM((1, _Q), jnp.float32),
            pltpu.VMEM((1, _Q), jnp.float32),
        ],
        compiler_params=pltpu.CompilerParams(
            dimension_semantics=("parallel", "arbitrary"),
        ),
    )(ptsT, points, points, at, ct, g1, b1, encT, g2, b2, wbigT, qb, ssel,
      clsT, clsb)
    return out.reshape(b, cls_w.shape[0])


# speedup vs baseline: 11.4530x; 5.3696x over previous
import functools
import math

import jax
import jax.numpy as jnp
from jax import lax
from jax.experimental import pallas as pl
from jax.experimental.pallas import tpu as pltpu

_B, _N, _K = 16, 2048, 20
_H1, _H2 = 64, 128
_Q = _H2 // 4
_EPS = 1e-5
_R = 256

_HIGH = lax.Precision.HIGHEST


def _dot(a, b):
    return lax.dot_general(a, b, (((1,), (0,)), ((), ())),
                           precision=_HIGH, preferred_element_type=jnp.float32)


def _fused_kernel(ptsT_ref, ptsR_ref, ptsN_ref, at_ref, ct_ref, g1_ref, b1_ref,
                  encT_ref, g2_ref, b2_ref, wbigT_ref, qb_ref, ssel_ref,
                  clsT_ref, clsb_ref, out_ref, amax_ref, asum_ref):
    t = pl.program_id(1)
    n_t = pl.num_programs(1)

    ptsT = ptsT_ref[0]
    Xi = ptsR_ref[0]
    ptsN = ptsN_ref[0]

    xxj = jnp.sum(ptsT * ptsT, axis=0, keepdims=True)
    xxi = jnp.sum(Xi * Xi, axis=1, keepdims=True)
    inner = -2.0 * lax.dot_general(
        Xi.astype(jnp.bfloat16), ptsT.astype(jnp.bfloat16),
        (((1,), (0,)), ((), ())), preferred_element_type=jnp.float32)
    P = -xxi - inner - xxj

    lane = lax.broadcasted_iota(jnp.int32, (_R, _N), 1)
    neg = jnp.float32(-jnp.inf)
    rid = lax.broadcasted_iota(jnp.int32, (_R, _N), 0) + t * _R
    P = jnp.where(lane == rid, neg, P)
    at = at_ref[...]
    u0 = _dot(Xi, at)
    Ub = _dot(ptsN, at).astype(jnp.bfloat16)
    Mx = u0
    Mn = u0
    for _ in range(_K - 1):
        am = jnp.argmax(P, axis=1)[:, None]
        sel = lane == am
        oh = jnp.where(sel, 1.0, 0.0).astype(jnp.bfloat16)
        u = lax.dot_general(oh, Ub, (((1,), (0,)), ((), ())),
                            preferred_element_type=jnp.float32)
        Mx = jnp.maximum(Mx, u)
        Mn = jnp.minimum(Mn, u)
        P = jnp.where(sel, neg, P)

    Vi = _dot(Xi, ct_ref[...])
    g1 = g1_ref[...]
    zx = g1 * (Mx + Vi) + b1_ref[...]
    zn = g1 * (Mn + Vi) + b1_ref[...]
    z = jnp.where(g1 >= 0.0, zx, zn)
    edge = jnp.where(z > 0.0, z, 0.2 * z)

    e = g2_ref[...] * _dot(edge, encT_ref[...]) + b2_ref[...]
    e = 0.5 * e * (1.0 + lax.erf(e * (1.0 / math.sqrt(2.0))))

    mixed = _dot(e, wbigT_ref[...]) + qb_ref[...]
    merged = _dot(mixed * mixed, ssel_ref[...])

    pmax = jnp.max(merged, axis=0, keepdims=True)
    psum = jnp.sum(merged, axis=0, keepdims=True)

    @pl.when(t == 0)
    def _():
        amax_ref[...] = pmax
        asum_ref[...] = psum

    @pl.when(t > 0)
    def _():
        amax_ref[...] = jnp.maximum(amax_ref[...], pmax)
        asum_ref[...] = asum_ref[...] + psum

    @pl.when(t == n_t - 1)
    def _():
        fmax = amax_ref[...]
        fmean = asum_ref[...] * (1.0 / _N)
        clsT = clsT_ref[...]
        logits = _dot(fmax, clsT[:_Q]) + _dot(fmean, clsT[_Q:]) + clsb_ref[...]
        out_ref[0] = logits


def kernel(inputs, edgeconv_w, bn1_gamma, bn1_beta, enc_w, bn2_gamma, bn2_beta,
           wr, wi, wj, wk, quat_bias, cls_w, cls_b):
    b = inputs.shape[0]
    points = inputs[..., :4].reshape(b, -1, 4)
    ptsT = jnp.transpose(points, (0, 2, 1))

    s1 = 1.0 / math.sqrt(1.0 + _EPS)
    at = edgeconv_w[:, :4].T
    ct = (edgeconv_w[:, 4:] - edgeconv_w[:, :4]).T
    g1 = (bn1_gamma * s1).reshape(1, _H1)
    b1 = bn1_beta.reshape(1, _H1)
    encT = enc_w.T
    g2 = (bn2_gamma * s1).reshape(1, _H2)
    b2 = bn2_beta.reshape(1, _H2)

    wbig = jnp.block([
        [wr, -wi, -wj, -wk],
        [wi, wr, wk, -wj],
        [wj, -wk, wr, wi],
        [wk, wj, -wi, wr],
    ])
    wbigT = wbig.T
    qb = quat_bias.reshape(1, _H2)

    ch = jnp.arange(_H2)
    ssel = (ch[:, None] // 4 == jnp.arange(_Q)[None, :]).astype(jnp.float32)

    clsT = cls_w.T
    clsb = cls_b.reshape(1, -1)

    n_t = _N // _R
    grid = (b, n_t)
    const = lambda *s: pl.BlockSpec(s, lambda bb, tt: (0,) * len(s))

    out = pl.pallas_call(
        _fused_kernel,
        grid=grid,
        in_specs=[
            pl.BlockSpec((1, 4, _N), lambda bb, tt: (bb, 0, 0)),
            pl.BlockSpec((1, _R, 4), lambda bb, tt: (bb, tt, 0)),
            pl.BlockSpec((1, _N, 4), lambda bb, tt: (bb, 0, 0)),
            const(4, _H1), const(4, _H1), const(1, _H1), const(1, _H1),
            const(_H1, _H2), const(1, _H2), const(1, _H2),
            const(_H2, _H2), const(1, _H2), const(_H2, _Q),
            const(2 * _Q, cls_w.shape[0]), const(1, cls_w.shape[0]),
        ],
        out_specs=pl.BlockSpec((1, 1, cls_w.shape[0]), lambda bb, tt: (bb, 0, 0)),
        out_shape=jax.ShapeDtypeStruct((b, 1, cls_w.shape[0]), jnp.float32),
        scratch_shapes=[
            pltpu.VMEM((1, _Q), jnp.float32),
            pltpu.VMEM((1, _Q), jnp.float32),
        ],
        compiler_params=pltpu.CompilerParams(
            dimension_semantics=("parallel", "arbitrary"),
        ),
    )(ptsT, points, points, at, ct, g1, b1, encT, g2, b2, wbigT, qb, ssel,
      clsT, clsb)
    return out.reshape(b, cls_w.shape[0])


# speedup vs baseline: 12.5817x; 1.0985x over previous
import functools
import math

import jax
import jax.numpy as jnp
from jax import lax
from jax.experimental import pallas as pl
from jax.experimental.pallas import tpu as pltpu
from jax.experimental.pallas import tpu_sc as plsc

_B, _N, _K = 16, 2048, 20
_H1, _H2 = 64, 128
_Q = _H2 // 4
_EPS = 1e-5
_R = 256
_D = 16
_CHUNK = 2048

_HIGH = lax.Precision.HIGHEST


def _dot(a, b):
    return lax.dot_general(a, b, (((1,), (0,)), ((), ())),
                           precision=_HIGH, preferred_element_type=jnp.float32)


def _knn_kernel(ptsT_ref, ptsR_ref, idx_ref):
    bb = pl.program_id(0)
    t = pl.program_id(1)
    ptsT = ptsT_ref[0]
    Xi = ptsR_ref[0]

    xxj = jnp.sum(ptsT * ptsT, axis=0, keepdims=True)
    xxi = jnp.sum(Xi * Xi, axis=1, keepdims=True)
    inner = -2.0 * lax.dot_general(
        Xi.astype(jnp.bfloat16), ptsT.astype(jnp.bfloat16),
        (((1,), (0,)), ((), ())), preferred_element_type=jnp.float32)
    P = -xxi - inner - xxj

    lane = lax.broadcasted_iota(jnp.int32, (_R, _N), 1)
    neg = jnp.float32(-jnp.inf)
    rid = lax.broadcasted_iota(jnp.int32, (_R, _N), 0) + t * _R
    P = jnp.where(lane == rid, neg, P)
    kiota = lax.broadcasted_iota(jnp.int32, (_R, _K), 1)
    cols = jnp.where(kiota == 0,
                     lax.broadcasted_iota(jnp.int32, (_R, _K), 0)
                     + (t * _R + bb * _N), 0)
    for k in range(1, _K):
        am = jnp.argmax(P, axis=1)[:, None]
        sel = lane == am
        cols = jnp.where(kiota == k, am + bb * _N, cols)
        P = jnp.where(sel, neg, P)
    idx_ref[...] = cols


def _edge_kernel(g_ref, ptsR_ref, at_ref, ct_ref, g1_ref, b1_ref,
                 encT_ref, g2_ref, b2_ref, wbigT_ref, qb_ref, ssel_ref,
                 clsT_ref, clsb_ref, out_ref, amax_ref, asum_ref):
    t = pl.program_id(1)
    n_t = pl.num_programs(1)
    Xi = ptsR_ref[0]
    at = at_ref[...]

    Mx = _dot(g_ref[0], at)
    Mn = Mx
    for k in range(1, _K):
        u = _dot(g_ref[k], at)
        Mx = jnp.maximum(Mx, u)
        Mn = jnp.minimum(Mn, u)

    Vi = _dot(Xi, ct_ref[...])
    g1 = g1_ref[...]
    zx = g1 * (Mx + Vi) + b1_ref[...]
    zn = g1 * (Mn + Vi) + b1_ref[...]
    z = jnp.where(g1 >= 0.0, zx, zn)
    edge = jnp.where(z > 0.0, z, 0.2 * z)

    e = g2_ref[...] * _dot(edge, encT_ref[...]) + b2_ref[...]
    e = 0.5 * e * (1.0 + lax.erf(e * (1.0 / math.sqrt(2.0))))

    mixed = _dot(e, wbigT_ref[...]) + qb_ref[...]
    merged = _dot(mixed * mixed, ssel_ref[...])

    pmax = jnp.max(merged, axis=0, keepdims=True)
    psum = jnp.sum(merged, axis=0, keepdims=True)

    @pl.when(t == 0)
    def _():
        amax_ref[...] = pmax
        asum_ref[...] = psum

    @pl.when(t > 0)
    def _():
        amax_ref[...] = jnp.maximum(amax_ref[...], pmax)
        asum_ref[...] = asum_ref[...] + psum

    @pl.when(t == n_t - 1)
    def _():
        fmax = amax_ref[...]
        fmean = asum_ref[...] * (1.0 / _N)
        clsT = clsT_ref[...]
        logits = _dot(fmax, clsT[:_Q]) + _dot(fmean, clsT[_Q:]) + clsb_ref[...]
        out_ref[0] = logits


def _make_sc_gather():
    info = plsc.get_sparse_core_info()
    nc, ns = info.num_cores, info.num_subcores
    nw = nc * ns
    total = _K * _B * _N
    per_w = total // nw
    n_chunks = per_w // _CHUNK
    mesh = plsc.VectorSubcoreMesh(core_axis_name="c", subcore_axis_name="s")

    @functools.partial(
        pl.kernel, mesh=mesh,
        out_type=jax.ShapeDtypeStruct((total, _D), jnp.float32),
        compiler_params=pltpu.CompilerParams(use_tc_tiling_on_sc=False),
        scratch_types=[
            pltpu.VMEM((_CHUNK,), jnp.int32),
            pltpu.VMEM((_CHUNK, _D), jnp.float32),
            pltpu.SemaphoreType.DMA,
        ],
    )
    def sc_gather(idx_hbm, table_hbm, out_hbm, idx_v, rows_v, sem):
        wid = lax.axis_index("s") * nc + lax.axis_index("c")
        base = wid * per_w
        for c in range(n_chunks):
            off = base + c * _CHUNK
            pltpu.sync_copy(idx_hbm.at[pl.ds(off, _CHUNK)], idx_v)
            pltpu.async_copy(table_hbm.at[idx_v], rows_v, sem).wait()
            pltpu.sync_copy(rows_v, out_hbm.at[pl.ds(off, _CHUNK)])

    return sc_gather


def kernel(inputs, edgeconv_w, bn1_gamma, bn1_beta, enc_w, bn2_gamma, bn2_beta,
           wr, wi, wj, wk, quat_bias, cls_w, cls_b):
    b = inputs.shape[0]
    points = inputs[..., :4].reshape(b, -1, 4)
    ptsT = jnp.transpose(points, (0, 2, 1))

    n_t = _N // _R
    idx = pl.pallas_call(
        _knn_kernel,
        grid=(b, n_t),
        in_specs=[
            pl.BlockSpec((1, 4, _N), lambda bb, tt: (bb, 0, 0)),
            pl.BlockSpec((1, _R, 4), lambda bb, tt: (bb, tt, 0)),
        ],
        out_specs=pl.BlockSpec((_R, _K), lambda bb, tt: (bb * n_t + tt, 0)),
        out_shape=jax.ShapeDtypeStruct((b * _N, _K), jnp.int32),
        compiler_params=pltpu.CompilerParams(
            dimension_semantics=("arbitrary", "arbitrary"),
        ),
    )(ptsT, points)

    idxT = idx.T.reshape(-1)
    table = jnp.pad(points.reshape(b * _N, 4), ((0, 0), (0, _D - 4)))

    gathered = _make_sc_gather()(idxT, table)
    gathered = gathered.reshape(_K, b * _N, _D)

    s1 = 1.0 / math.sqrt(1.0 + _EPS)
    at = jnp.pad(edgeconv_w[:, :4].T, ((0, _D - 4), (0, 0)))
    ct = (edgeconv_w[:, 4:] - edgeconv_w[:, :4]).T
    g1 = (bn1_gamma * s1).reshape(1, _H1)
    b1 = bn1_beta.reshape(1, _H1)
    encT = enc_w.T
    g2 = (bn2_gamma * s1).reshape(1, _H2)
    b2 = bn2_beta.reshape(1, _H2)
    wbig = jnp.block([
        [wr, -wi, -wj, -wk],
        [wi, wr, wk, -wj],
        [wj, -wk, wr, wi],
        [wk, wj, -wi, wr],
    ])
    wbigT = wbig.T
    qb = quat_bias.reshape(1, _H2)
    ch = jnp.arange(_H2)
    ssel = (ch[:, None] // 4 == jnp.arange(_Q)[None, :]).astype(jnp.float32)
    clsT = cls_w.T
    clsb = cls_b.reshape(1, -1)

    const = lambda *s: pl.BlockSpec(s, lambda bb, tt: (0,) * len(s))
    out = pl.pallas_call(
        _edge_kernel,
        grid=(b, n_t),
        in_specs=[
            pl.BlockSpec((_K, _R, _D), lambda bb, tt: (0, bb * n_t + tt, 0)),
            pl.BlockSpec((1, _R, 4), lambda bb, tt: (bb, tt, 0)),
            const(_D, _H1), const(4, _H1), const(1, _H1), const(1, _H1),
            const(_H1, _H2), const(1, _H2), const(1, _H2),
            const(_H2, _H2), const(1, _H2), const(_H2, _Q),
            const(2 * _Q, cls_w.shape[0]), const(1, cls_w.shape[0]),
        ],
        out_specs=pl.BlockSpec((1, 1, cls_w.shape[0]), lambda bb, tt: (bb, 0, 0)),
        out_shape=jax.ShapeDtypeStruct((b, 1, cls_w.shape[0]), jnp.float32),
        scratch_shapes=[
            pltpu.VMEM((1, _Q), jnp.float32),
            pltpu.VMEM((1, _Q), jnp.float32),
        ],
        compiler_params=pltpu.CompilerParams(
            dimension_semantics=("arbitrary", "arbitrary"),
        ),
    )(gathered, points, at, ct, g1, b1, encT, g2, b2, wbigT, qb, ssel,
      clsT, clsb)
    return out.reshape(b, cls_w.shape[0])


# speedup vs baseline: 12.8048x; 1.0177x over previous
import functools
import math

import jax
import jax.numpy as jnp
from jax import lax
from jax.experimental import pallas as pl
from jax.experimental.pallas import tpu as pltpu
from jax.experimental.pallas import tpu_sc as plsc

_B, _N, _K = 16, 2048, 20
_H1, _H2 = 64, 128
_Q = _H2 // 4
_EPS = 1e-5
_R = 512
_CHUNK = 2048

_HIGH = lax.Precision.HIGHEST


def _dot(a, b):
    return lax.dot_general(a, b, (((1,), (0,)), ((), ())),
                           precision=_HIGH, preferred_element_type=jnp.float32)


def _knn_kernel(ptsT_ref, ptsR_ref, at_ref, idx_ref, u_ref):
    bb = pl.program_id(0)
    t = pl.program_id(1)
    ptsT = ptsT_ref[0]
    Xi = ptsR_ref[0]

    u_ref[...] = _dot(Xi, at_ref[...]).astype(jnp.bfloat16)

    xxj = jnp.sum(ptsT * ptsT, axis=0, keepdims=True)
    xxi = jnp.sum(Xi * Xi, axis=1, keepdims=True)
    inner = -2.0 * lax.dot_general(
        Xi.astype(jnp.bfloat16), ptsT.astype(jnp.bfloat16),
        (((1,), (0,)), ((), ())), preferred_element_type=jnp.float32)
    P = -xxi - inner - xxj

    lane = lax.broadcasted_iota(jnp.int32, (_R, _N), 1)
    neg = jnp.float32(-jnp.inf)
    rid = lax.broadcasted_iota(jnp.int32, (_R, _N), 0) + t * _R
    P = jnp.where(lane == rid, neg, P)
    kiota = lax.broadcasted_iota(jnp.int32, (_R, _K), 1)
    cols = jnp.where(kiota == 0,
                     lax.broadcasted_iota(jnp.int32, (_R, _K), 0)
                     + (t * _R + bb * _N), 0)
    for k in range(1, _K):
        am = jnp.argmax(P, axis=1)[:, None]
        sel = lane == am
        cols = jnp.where(kiota == k, am + bb * _N, cols)
        P = jnp.where(sel, neg, P)
    idx_ref[...] = cols


def _edge_kernel(g_ref, ptsR_ref, ct_ref, g1_ref, b1_ref,
                 encT_ref, g2_ref, b2_ref, wbigT_ref, qb_ref, ssel_ref,
                 clsT_ref, clsb_ref, out_ref, amax_ref, asum_ref):
    t = pl.program_id(1)
    n_t = pl.num_programs(1)
    Xi = ptsR_ref[0]

    Mxb = g_ref[0]
    Mnb = Mxb
    for k in range(1, _K):
        u = g_ref[k]
        Mxb = jnp.maximum(Mxb, u)
        Mnb = jnp.minimum(Mnb, u)
    Mx = Mxb.astype(jnp.float32)
    Mn = Mnb.astype(jnp.float32)

    Vi = _dot(Xi, ct_ref[...])
    g1 = g1_ref[...]
    zx = g1 * (Mx + Vi) + b1_ref[...]
    zn = g1 * (Mn + Vi) + b1_ref[...]
    z = jnp.where(g1 >= 0.0, zx, zn)
    edge = jnp.where(z > 0.0, z, 0.2 * z)

    e = g2_ref[...] * _dot(edge, encT_ref[...]) + b2_ref[...]
    e = 0.5 * e * (1.0 + lax.erf(e * (1.0 / math.sqrt(2.0))))

    mixed = _dot(e, wbigT_ref[...]) + qb_ref[...]
    merged = _dot(mixed * mixed, ssel_ref[...])

    pmax = jnp.max(merged, axis=0, keepdims=True)
    psum = jnp.sum(merged, axis=0, keepdims=True)

    @pl.when(t == 0)
    def _():
        amax_ref[...] = pmax
        asum_ref[...] = psum

    @pl.when(t > 0)
    def _():
        amax_ref[...] = jnp.maximum(amax_ref[...], pmax)
        asum_ref[...] = asum_ref[...] + psum

    @pl.when(t == n_t - 1)
    def _():
        fmax = amax_ref[...]
        fmean = asum_ref[...] * (1.0 / _N)
        clsT = clsT_ref[...]
        logits = _dot(fmax, clsT[:_Q]) + _dot(fmean, clsT[_Q:]) + clsb_ref[...]
        out_ref[0] = logits


def _make_sc_gather(total):
    info = plsc.get_sparse_core_info()
    nc, ns = info.num_cores, info.num_subcores
    nw = nc * ns
    per_w = total // nw
    n_chunks = per_w // _CHUNK
    mesh = plsc.VectorSubcoreMesh(core_axis_name="c", subcore_axis_name="s")

    @functools.partial(
        pl.kernel, mesh=mesh,
        out_type=jax.ShapeDtypeStruct((total, _H1), jnp.bfloat16),
        compiler_params=pltpu.CompilerParams(use_tc_tiling_on_sc=False),
        scratch_types=[
            pltpu.VMEM((_CHUNK,), jnp.int32),
            pltpu.VMEM((_CHUNK, _H1), jnp.bfloat16),
            pltpu.SemaphoreType.DMA,
        ],
    )
    def sc_gather(idx_hbm, table_hbm, out_hbm, idx_v, rows_v, sem):
        wid = lax.axis_index("s") * nc + lax.axis_index("c")
        base = wid * per_w
        for c in range(n_chunks):
            off = base + c * _CHUNK
            pltpu.sync_copy(idx_hbm.at[pl.ds(off, _CHUNK)], idx_v)
            pltpu.async_copy(table_hbm.at[idx_v], rows_v, sem).wait()
            pltpu.sync_copy(rows_v, out_hbm.at[pl.ds(off, _CHUNK)])

    return sc_gather


def kernel(inputs, edgeconv_w, bn1_gamma, bn1_beta, enc_w, bn2_gamma, bn2_beta,
           wr, wi, wj, wk, quat_bias, cls_w, cls_b):
    b = inputs.shape[0]
    points = inputs[..., :4].reshape(b, -1, 4)
    ptsT = jnp.transpose(points, (0, 2, 1))
    at = edgeconv_w[:, :4].T

    n_t = _N // _R
    idx, utab = pl.pallas_call(
        _knn_kernel,
        grid=(b, n_t),
        in_specs=[
            pl.BlockSpec((1, 4, _N), lambda bb, tt: (bb, 0, 0)),
            pl.BlockSpec((1, _R, 4), lambda bb, tt: (bb, tt, 0)),
            pl.BlockSpec((4, _H1), lambda bb, tt: (0, 0)),
        ],
        out_specs=[
            pl.BlockSpec((_R, _K), lambda bb, tt: (bb * n_t + tt, 0)),
            pl.BlockSpec((_R, _H1), lambda bb, tt: (bb * n_t + tt, 0)),
        ],
        out_shape=[
            jax.ShapeDtypeStruct((b * _N, _K), jnp.int32),
            jax.ShapeDtypeStruct((b * _N, _H1), jnp.bfloat16),
        ],
        compiler_params=pltpu.CompilerParams(
            dimension_semantics=("arbitrary", "arbitrary"),
        ),
    )(ptsT, points, at)

    idxT = idx.T.reshape(-1)
    gathered = _make_sc_gather(_K * b * _N)(idxT, utab)
    gathered = gathered.reshape(_K, b * _N, _H1)

    s1 = 1.0 / math.sqrt(1.0 + _EPS)
    ct = (edgeconv_w[:, 4:] - edgeconv_w[:, :4]).T
    g1 = (bn1_gamma * s1).reshape(1, _H1)
    b1 = bn1_beta.reshape(1, _H1)
    encT = enc_w.T
    g2 = (bn2_gamma * s1).reshape(1, _H2)
    b2 = bn2_beta.reshape(1, _H2)
    wbig = jnp.block([
        [wr, -wi, -wj, -wk],
        [wi, wr, wk, -wj],
        [wj, -wk, wr, wi],
        [wk, wj, -wi, wr],
    ])
    wbigT = wbig.T
    qb = quat_bias.reshape(1, _H2)
    ch = jnp.arange(_H2)
    ssel = (ch[:, None] // 4 == jnp.arange(_Q)[None, :]).astype(jnp.float32)
    clsT = cls_w.T
    clsb = cls_b.reshape(1, -1)

    const = lambda *s: pl.BlockSpec(s, lambda bb, tt: (0,) * len(s))
    out = pl.pallas_call(
        _edge_kernel,
        grid=(b, n_t),
        in_specs=[
            pl.BlockSpec((_K, _R, _H1), lambda bb, tt: (0, bb * n_t + tt, 0)),
            pl.BlockSpec((1, _R, 4), lambda bb, tt: (bb, tt, 0)),
            const(4, _H1), const(1, _H1), const(1, _H1),
            const(_H1, _H2), const(1, _H2), const(1, _H2),
            const(_H2, _H2), const(1, _H2), const(_H2, _Q),
            const(2 * _Q, cls_w.shape[0]), const(1, cls_w.shape[0]),
        ],
        out_specs=pl.BlockSpec((1, 1, cls_w.shape[0]), lambda bb, tt: (bb, 0, 0)),
        out_shape=jax.ShapeDtypeStruct((b, 1, cls_w.shape[0]), jnp.float32),
        scratch_shapes=[
            pltpu.VMEM((1, _Q), jnp.float32),
            pltpu.VMEM((1, _Q), jnp.float32),
        ],
        compiler_params=pltpu.CompilerParams(
            dimension_semantics=("arbitrary", "arbitrary"),
        ),
    )(gathered, points, ct, g1, b1, encT, g2, b2, wbigT, qb, ssel,
      clsT, clsb)
    return out.reshape(b, cls_w.shape[0])


# speedup vs baseline: 12.8948x; 1.0070x over previous
import functools
import math

import jax
import jax.numpy as jnp
from jax import lax
from jax.experimental import pallas as pl
from jax.experimental.pallas import tpu as pltpu
from jax.experimental.pallas import tpu_sc as plsc

_B, _N, _K = 16, 2048, 20
_H1, _H2 = 64, 128
_Q = _H2 // 4
_EPS = 1e-5
_R = 512
_CHUNK = 2048

_HIGH = lax.Precision.HIGHEST


def _dot(a, b):
    return lax.dot_general(a, b, (((1,), (0,)), ((), ())),
                           precision=_HIGH, preferred_element_type=jnp.float32)


def _knn_kernel(ptsT_ref, ptsR_ref, at_ref, idx_ref, u_ref):
    bb = pl.program_id(0)
    t = pl.program_id(1)
    ptsT = ptsT_ref[0]
    Xi = ptsR_ref[0]

    u_ref[...] = _dot(Xi, at_ref[...]).astype(jnp.bfloat16)

    xxj = jnp.sum(ptsT * ptsT, axis=0, keepdims=True)
    xxi = jnp.sum(Xi * Xi, axis=1, keepdims=True)
    inner = -2.0 * lax.dot_general(
        Xi.astype(jnp.bfloat16), ptsT.astype(jnp.bfloat16),
        (((1,), (0,)), ((), ())), preferred_element_type=jnp.float32)
    P = -xxi - inner - xxj

    lane = lax.broadcasted_iota(jnp.int32, (_R, _N), 1)
    neg = jnp.float32(-jnp.inf)
    rid = lax.broadcasted_iota(jnp.int32, (_R, _N), 0) + t * _R
    P = jnp.where(lane == rid, neg, P)
    kiota = lax.broadcasted_iota(jnp.int32, (_R, _K), 1)
    cols = jnp.where(kiota == 0,
                     lax.broadcasted_iota(jnp.int32, (_R, _K), 0)
                     + (t * _R + bb * _N), 0)
    for k in range(1, _K):
        am = jnp.argmax(P, axis=1)[:, None]
        sel = lane == am
        cols = jnp.where(kiota == k, am + bb * _N, cols)
        P = jnp.where(sel, neg, P)
    idx_ref[...] = cols


def _edge_kernel(g_ref, ptsR_ref, ct_ref, g1_ref, b1_ref,
                 encT_ref, g2_ref, b2_ref, wbigT_ref, qb_ref, ssel_ref,
                 clsT_ref, clsb_ref, out_ref, amax_ref, asum_ref):
    t = pl.program_id(1)
    n_t = pl.num_programs(1)
    Xi = ptsR_ref[0]

    Mxb = g_ref[0]
    Mnb = Mxb
    for k in range(1, _K):
        u = g_ref[k]
        Mxb = jnp.maximum(Mxb, u)
        Mnb = jnp.minimum(Mnb, u)
    Mx = Mxb.astype(jnp.float32)
    Mn = Mnb.astype(jnp.float32)

    Vi = _dot(Xi, ct_ref[...])
    g1 = g1_ref[...]
    zx = g1 * (Mx + Vi) + b1_ref[...]
    zn = g1 * (Mn + Vi) + b1_ref[...]
    z = jnp.where(g1 >= 0.0, zx, zn)
    edge = jnp.where(z > 0.0, z, 0.2 * z)

    e = g2_ref[...] * _dot(edge, encT_ref[...]) + b2_ref[...]
    e = 0.5 * e * (1.0 + lax.erf(e * (1.0 / math.sqrt(2.0))))

    mixed = _dot(e, wbigT_ref[...]) + qb_ref[...]
    merged = _dot(mixed * mixed, ssel_ref[...])

    pmax = jnp.max(merged, axis=0, keepdims=True)
    psum = jnp.sum(merged, axis=0, keepdims=True)

    @pl.when(t == 0)
    def _():
        amax_ref[...] = pmax
        asum_ref[...] = psum

    @pl.when(t > 0)
    def _():
        amax_ref[...] = jnp.maximum(amax_ref[...], pmax)
        asum_ref[...] = asum_ref[...] + psum

    @pl.when(t == n_t - 1)
    def _():
        fmax = amax_ref[...]
        fmean = asum_ref[...] * (1.0 / _N)
        clsT = clsT_ref[...]
        logits = _dot(fmax, clsT[:_Q]) + _dot(fmean, clsT[_Q:]) + clsb_ref[...]
        out_ref[0] = logits


def _make_sc_gather(total):
    info = plsc.get_sparse_core_info()
    nc, ns = info.num_cores, info.num_subcores
    nw = nc * ns
    per_w = total // nw
    n_chunks = per_w // _CHUNK
    mesh = plsc.VectorSubcoreMesh(core_axis_name="c", subcore_axis_name="s")

    @functools.partial(
        pl.kernel, mesh=mesh,
        out_type=jax.ShapeDtypeStruct((total, _H1), jnp.bfloat16),
        compiler_params=pltpu.CompilerParams(use_tc_tiling_on_sc=False),
        scratch_types=[
            pltpu.VMEM((_CHUNK,), jnp.int32),
            pltpu.VMEM((_CHUNK, _H1), jnp.bfloat16),
            pltpu.SemaphoreType.DMA,
        ],
    )
    def sc_gather(idx_hbm, table_hbm, out_hbm, idx_v, rows_v, sem):
        wid = lax.axis_index("s") * nc + lax.axis_index("c")
        base = wid * per_w
        for c in range(n_chunks):
            off = base + c * _CHUNK
            pltpu.sync_copy(idx_hbm.at[pl.ds(off, _CHUNK)], idx_v)
            pltpu.async_copy(table_hbm.at[idx_v], rows_v, sem).wait()
            pltpu.sync_copy(rows_v, out_hbm.at[pl.ds(off, _CHUNK)])

    return sc_gather


def _run_knn(ptsT_h, pts_h, at, bh):
    n_t = _N // _R
    return pl.pallas_call(
        _knn_kernel,
        grid=(bh, n_t),
        in_specs=[
            pl.BlockSpec((1, 4, _N), lambda bb, tt: (bb, 0, 0)),
            pl.BlockSpec((1, _R, 4), lambda bb, tt: (bb, tt, 0)),
            pl.BlockSpec((4, _H1), lambda bb, tt: (0, 0)),
        ],
        out_specs=[
            pl.BlockSpec((_R, _K), lambda bb, tt: (bb * n_t + tt, 0)),
            pl.BlockSpec((_R, _H1), lambda bb, tt: (bb * n_t + tt, 0)),
        ],
        out_shape=[
            jax.ShapeDtypeStruct((bh * _N, _K), jnp.int32),
            jax.ShapeDtypeStruct((bh * _N, _H1), jnp.bfloat16),
        ],
        compiler_params=pltpu.CompilerParams(
            dimension_semantics=("arbitrary", "arbitrary"),
        ),
    )(ptsT_h, pts_h, at)


def kernel(inputs, edgeconv_w, bn1_gamma, bn1_beta, enc_w, bn2_gamma, bn2_beta,
           wr, wi, wj, wk, quat_bias, cls_w, cls_b):
    b = inputs.shape[0]
    points = inputs[..., :4].reshape(b, -1, 4)
    ptsT = jnp.transpose(points, (0, 2, 1))
    at = edgeconv_w[:, :4].T

    nh = 2
    bh = b // nh
    gathered_halves = []
    for h in range(nh):
        idx_h, utab_h = _run_knn(ptsT[h * bh:(h + 1) * bh],
                                 points[h * bh:(h + 1) * bh], at, bh)
        idxT_h = idx_h.T.reshape(-1)
        g_h = _make_sc_gather(_K * bh * _N)(idxT_h, utab_h)
        gathered_halves.append(g_h.reshape(_K, bh * _N, _H1))

    s1 = 1.0 / math.sqrt(1.0 + _EPS)
    ct = (edgeconv_w[:, 4:] - edgeconv_w[:, :4]).T
    g1 = (bn1_gamma * s1).reshape(1, _H1)
    b1 = bn1_beta.reshape(1, _H1)
    encT = enc_w.T
    g2 = (bn2_gamma * s1).reshape(1, _H2)
    b2 = bn2_beta.reshape(1, _H2)
    wbig = jnp.block([
        [wr, -wi, -wj, -wk],
        [wi, wr, wk, -wj],
        [wj, -wk, wr, wi],
        [wk, wj, -wi, wr],
    ])
    wbigT = wbig.T
    qb = quat_bias.reshape(1, _H2)
    ch = jnp.arange(_H2)
    ssel = (ch[:, None] // 4 == jnp.arange(_Q)[None, :]).astype(jnp.float32)
    clsT = cls_w.T
    clsb = cls_b.reshape(1, -1)

    const = lambda *s: pl.BlockSpec(s, lambda bb, tt: (0,) * len(s))
    n_t = _N // _R
    outs = []
    for h in range(nh):
        out_h = pl.pallas_call(
            _edge_kernel,
            grid=(bh, n_t),
            in_specs=[
                pl.BlockSpec((_K, _R, _H1),
                             lambda bb, tt: (0, bb * n_t + tt, 0)),
                pl.BlockSpec((1, _R, 4), lambda bb, tt: (bb, tt, 0)),
                const(4, _H1), const(1, _H1), const(1, _H1),
                const(_H1, _H2), const(1, _H2), const(1, _H2),
                const(_H2, _H2), const(1, _H2), const(_H2, _Q),
                const(2 * _Q, cls_w.shape[0]), const(1, cls_w.shape[0]),
            ],
            out_specs=pl.BlockSpec((1, 1, cls_w.shape[0]),
                                   lambda bb, tt: (bb, 0, 0)),
            out_shape=jax.ShapeDtypeStruct((bh, 1, cls_w.shape[0]),
                                           jnp.float32),
            scratch_shapes=[
                pltpu.VMEM((1, _Q), jnp.float32),
                pltpu.VMEM((1, _Q), jnp.float32),
            ],
            compiler_params=pltpu.CompilerParams(
                dimension_semantics=("arbitrary", "arbitrary"),
            ),
        )(gathered_halves[h], points[h * bh:(h + 1) * bh], ct, g1, b1,
          encT, g2, b2, wbigT, qb, ssel, clsT, clsb)
        outs.append(out_h.reshape(bh, cls_w.shape[0]))
    return jnp.concatenate(outs, axis=0)
